# baseline (speedup 1.0000x reference)
import functools

import jax
import jax.numpy as jnp
from jax import lax
from jax.experimental import pallas as pl
from jax.experimental.pallas import tpu as pltpu
from jax.experimental.pallas import tpu_sc as plsc

NC = 2
NS = 16
L = 16
NW = NC * NS

B = 16384
BPW = B // NW
NCHUNK = BPW // L

CONT_PARTS = ((13, 0), (3, 13), (13, 16))
EMB_PARTS = ((32, 29), (4, 61), (8, 65), (2, 73),
             (2, 75), (4, 77), (8, 81))
D_OUT = 89

_TABLE_SHAPES = ((400, 32), (8, 4), (33, 8), (3, 2), (3, 2), (32, 4), (64, 8))

_mesh = plsc.VectorSubcoreMesh(
    core_axis_name="c", subcore_axis_name="s", num_cores=NC, num_subcores=NS)


@functools.partial(
    pl.kernel,
    mesh=_mesh,
    compiler_params=pltpu.CompilerParams(
        needs_layout_passes=False, use_tc_tiling_on_sc=False),
    out_type=jax.ShapeDtypeStruct((B, D_OUT), jnp.float32),
    scratch_types=(
        [pltpu.VMEM((BPW, 13), jnp.float32),
         pltpu.VMEM((BPW, 3), jnp.float32),
         pltpu.VMEM((BPW, 13), jnp.float32)]
        + [pltpu.VMEM((BPW,), jnp.int32) for _ in range(7)]
        + [pltpu.VMEM(s, jnp.float32) for s in _TABLE_SHAPES]
        + [pltpu.VMEM((BPW, D_OUT), jnp.float32)]
        + [pltpu.SemaphoreType.DMA]
    ),
)
def _encode(cont_h, bin_h, ctrl_h,
            act_h, jmp_h, chr_h, lc_h, hb_h, gnd_h, la_h,
            wa_h, wj_h, wc_h, wl_h, wh_h, wg_h, wla_h,
            out_h,
            cont_v, bin_v, ctrl_v,
            act_v, jmp_v, chr_v, lc_v, hb_v, gnd_v, la_v,
            wa_v, wj_v, wc_v, wl_v, wh_v, wg_v, wla_v,
            out_v, dma_sem):
  wid = lax.axis_index("s") * NC + lax.axis_index("c")
  base = wid * BPW

  idx_refs = (act_v, jmp_v, chr_v, lc_v, hb_v, gnd_v, la_v)
  tbl_refs = (wa_v, wj_v, wc_v, wl_v, wh_v, wg_v, wla_v)
  copies = []
  for h, v in ((cont_h, cont_v), (bin_h, bin_v), (ctrl_h, ctrl_v)):
    copies.append(pltpu.async_copy(h.at[pl.ds(base, BPW)], v, dma_sem))
  for h, v in zip((act_h, jmp_h, chr_h, lc_h, hb_h, gnd_h, la_h), idx_refs):
    copies.append(pltpu.async_copy(h.at[pl.ds(base, BPW)], v, dma_sem))
  for h, v in zip((wa_h, wj_h, wc_h, wl_h, wh_h, wg_h, wla_h), tbl_refs):
    copies.append(pltpu.async_copy(h, v, dma_sem))
  for c in copies:
    c.wait()

  lane = lax.iota(jnp.int32, L)
  GRP = 8

  def chunk(k, carry):
    rows = lane + k * L
    jobs = []
    for src, (w, off) in zip((cont_v, bin_v, ctrl_v), CONT_PARTS):
      for c in range(w):
        jobs.append((src, rows, c, off + c))
    for iv, tv, (w, off) in zip(idx_refs, tbl_refs, EMB_PARTS):
      idx = iv[pl.ds(k * L, L)]
      for c in range(w):
        jobs.append((tv, idx, c, off + c))
    for g in range(0, len(jobs), GRP):
      grp = jobs[g:g + GRP]
      vals = [plsc.load_gather(src, [ri, jnp.full((L,), c, jnp.int32)])
              for src, ri, c, _ in grp]
      for (_, _, _, oc), v in zip(grp, vals):
        plsc.store_scatter(out_v, [rows, jnp.full((L,), oc, jnp.int32)], v)
    return carry

  lax.fori_loop(0, NCHUNK, chunk, 0)
  pltpu.sync_copy(out_v, out_h.at[pl.ds(base, BPW)])


def kernel(continuous, binary, controller, action, jumps_left, character,
           l_cancel, hurtbox_state, ground, last_attack_landed,
           W_action, W_jumps, W_character, W_l_cancel, W_hurtbox, W_ground,
           W_last_attack):
  to_i32 = lambda x: x.astype(jnp.int32)
  return _encode(continuous, binary, controller,
                 to_i32(action), to_i32(jumps_left), to_i32(character),
                 to_i32(l_cancel), to_i32(hurtbox_state), to_i32(ground),
                 to_i32(last_attack_landed),
                 W_action, W_jumps, W_character, W_l_cancel, W_hurtbox,
                 W_ground, W_last_attack)


# speedup vs baseline: 1.0367x; 1.0367x over previous
import functools

import jax
import jax.numpy as jnp
from jax import lax
from jax.experimental import pallas as pl
from jax.experimental.pallas import tpu as pltpu
from jax.experimental.pallas import tpu_sc as plsc

NC = 2
NS = 16
L = 16
NW = NC * NS

B = 16384
BPW = B // NW
PIECE = 64
NPIECE = BPW // PIECE
NCHUNK = PIECE // L

CONT_PARTS = ((13, 0), (3, 13), (13, 16))
EMB_PARTS = ((32, 29), (4, 61), (8, 65), (2, 73),
             (2, 75), (4, 77), (8, 81))
D_OUT = 89

_TABLE_SHAPES = ((400, 32), (8, 4), (33, 8), (3, 2), (3, 2), (32, 4), (64, 8))

_mesh = plsc.VectorSubcoreMesh(
    core_axis_name="c", subcore_axis_name="s", num_cores=NC, num_subcores=NS)


@functools.partial(
    pl.kernel,
    mesh=_mesh,
    compiler_params=pltpu.CompilerParams(
        needs_layout_passes=False, use_tc_tiling_on_sc=True),
    out_type=jax.ShapeDtypeStruct((B, D_OUT), jnp.float32),
    scratch_types=(
        [pltpu.VMEM((PIECE, 13), jnp.float32),
         pltpu.VMEM((PIECE, 3), jnp.float32),
         pltpu.VMEM((PIECE, 13), jnp.float32)]
        + [pltpu.VMEM((BPW,), jnp.int32) for _ in range(7)]
        + [pltpu.VMEM(s, jnp.float32) for s in _TABLE_SHAPES]
        + [pltpu.VMEM((PIECE, D_OUT), jnp.float32)]
        + [pltpu.SemaphoreType.DMA]
    ),
)
def _encode(cont_h, bin_h, ctrl_h,
            act_h, jmp_h, chr_h, lc_h, hb_h, gnd_h, la_h,
            wa_h, wj_h, wc_h, wl_h, wh_h, wg_h, wla_h,
            out_h,
            cont_v, bin_v, ctrl_v,
            act_v, jmp_v, chr_v, lc_v, hb_v, gnd_v, la_v,
            wa_v, wj_v, wc_v, wl_v, wh_v, wg_v, wla_v,
            out_v, dma_sem):
  wid = lax.axis_index("s") * NC + lax.axis_index("c")
  base = wid * BPW

  idx_refs = (act_v, jmp_v, chr_v, lc_v, hb_v, gnd_v, la_v)
  tbl_refs = (wa_v, wj_v, wc_v, wl_v, wh_v, wg_v, wla_v)
  copies = []
  for h, v in zip((act_h, jmp_h, chr_h, lc_h, hb_h, gnd_h, la_h), idx_refs):
    copies.append(pltpu.async_copy(h.at[pl.ds(base, BPW)], v, dma_sem))
  for h, v in zip((wa_h, wj_h, wc_h, wl_h, wh_h, wg_h, wla_h), tbl_refs):
    copies.append(pltpu.async_copy(h, v, dma_sem))
  for c in copies:
    c.wait()

  lane = lax.iota(jnp.int32, L)
  GRP = 8

  def piece(p, carry):
    pbase = base + p * PIECE
    for h, v in ((cont_h, cont_v), (bin_h, bin_v), (ctrl_h, ctrl_v)):
      pltpu.sync_copy(h.at[pl.ds(pbase, PIECE)], v)

    def chunk(k, carry2):
      rows = lane + k * L
      jobs = []
      for src, (w, off) in zip((cont_v, bin_v, ctrl_v), CONT_PARTS):
        for c in range(w):
          jobs.append((src, rows, c, off + c))
      for iv, tv, (w, off) in zip(idx_refs, tbl_refs, EMB_PARTS):
        idx = iv[pl.ds(p * PIECE + k * L, L)]
        for c in range(w):
          jobs.append((tv, idx, c, off + c))
      for g in range(0, len(jobs), GRP):
        grp = jobs[g:g + GRP]
        vals = [plsc.load_gather(src, [ri, jnp.full((L,), c, jnp.int32)])
                for src, ri, c, _ in grp]
        for (_, _, _, oc), v in zip(grp, vals):
          plsc.store_scatter(out_v, [rows, jnp.full((L,), oc, jnp.int32)], v)
      return carry2

    lax.fori_loop(0, NCHUNK, chunk, 0)
    pltpu.sync_copy(out_v, out_h.at[pl.ds(pbase, PIECE)])
    return carry

  lax.fori_loop(0, NPIECE, piece, 0)


def kernel(continuous, binary, controller, action, jumps_left, character,
           l_cancel, hurtbox_state, ground, last_attack_landed,
           W_action, W_jumps, W_character, W_l_cancel, W_hurtbox, W_ground,
           W_last_attack):
  to_i32 = lambda x: x.astype(jnp.int32)
  return _encode(continuous, binary, controller,
                 to_i32(action), to_i32(jumps_left), to_i32(character),
                 to_i32(l_cancel), to_i32(hurtbox_state), to_i32(ground),
                 to_i32(last_attack_landed),
                 W_action, W_jumps, W_character, W_l_cancel, W_hurtbox,
                 W_ground, W_last_attack)


# speedup vs baseline: 4.0919x; 3.9472x over previous
import functools

import jax
import jax.numpy as jnp
from jax import lax
from jax.experimental import pallas as pl
from jax.experimental.pallas import tpu as pltpu
from jax.experimental.pallas import tpu_sc as plsc

NC = 2
NS = 16
L = 16
NW = NC * NS

B = 16384
BPW = B // NW
NCHUNK = BPW // L

CONT_PARTS = ((13, 0), (3, 13), (13, 16))
EMB_PARTS = ((32, 29), (4, 61), (8, 65), (2, 73),
             (2, 75), (4, 77), (8, 81))
D_OUT = 89

_TABLE_SHAPES_T = ((32, 400), (4, 8), (8, 33), (2, 3), (2, 3), (4, 32), (8, 64))

_mesh = plsc.VectorSubcoreMesh(
    core_axis_name="c", subcore_axis_name="s", num_cores=NC, num_subcores=NS)


@functools.partial(
    pl.kernel,
    mesh=_mesh,
    compiler_params=pltpu.CompilerParams(
        needs_layout_passes=False, use_tc_tiling_on_sc=True),
    out_type=jax.ShapeDtypeStruct((D_OUT, B), jnp.float32),
    scratch_types=(
        [pltpu.VMEM((13, BPW), jnp.float32),
         pltpu.VMEM((3, BPW), jnp.float32),
         pltpu.VMEM((13, BPW), jnp.float32)]
        + [pltpu.VMEM((BPW,), jnp.int32) for _ in range(7)]
        + [pltpu.VMEM(s, jnp.float32) for s in _TABLE_SHAPES_T]
        + [pltpu.VMEM((D_OUT, BPW), jnp.float32)]
        + [pltpu.SemaphoreType.DMA]
    ),
)
def _encode(cont_h, bin_h, ctrl_h,
            act_h, jmp_h, chr_h, lc_h, hb_h, gnd_h, la_h,
            wa_h, wj_h, wc_h, wl_h, wh_h, wg_h, wla_h,
            out_h,
            cont_v, bin_v, ctrl_v,
            act_v, jmp_v, chr_v, lc_v, hb_v, gnd_v, la_v,
            wa_v, wj_v, wc_v, wl_v, wh_v, wg_v, wla_v,
            out_v, dma_sem):
  wid = lax.axis_index("s") * NC + lax.axis_index("c")
  base = wid * BPW

  idx_refs = (act_v, jmp_v, chr_v, lc_v, hb_v, gnd_v, la_v)
  tbl_refs = (wa_v, wj_v, wc_v, wl_v, wh_v, wg_v, wla_v)
  copies = []
  for h, v in ((cont_h, cont_v), (bin_h, bin_v), (ctrl_h, ctrl_v)):
    copies.append(pltpu.async_copy(h.at[:, pl.ds(base, BPW)], v, dma_sem))
  for h, v in zip((act_h, jmp_h, chr_h, lc_h, hb_h, gnd_h, la_h), idx_refs):
    copies.append(pltpu.async_copy(h.at[pl.ds(base, BPW)], v, dma_sem))
  for h, v in zip((wa_h, wj_h, wc_h, wl_h, wh_h, wg_h, wla_h), tbl_refs):
    copies.append(pltpu.async_copy(h, v, dma_sem))
  for c in copies:
    c.wait()

  GRP = 8

  def chunk(k, carry):
    cols = pl.ds(k * L, L)
    jobs = []
    for src, (w, off) in zip((cont_v, bin_v, ctrl_v), CONT_PARTS):
      for r in range(w):
        jobs.append((src, r, None, off + r))
    for iv, tv, (w, off) in zip(idx_refs, tbl_refs, EMB_PARTS):
      idx = iv[cols]
      for r in range(w):
        jobs.append((tv, r, idx, off + r))
    for g in range(0, len(jobs), GRP):
      grp = jobs[g:g + GRP]
      vals = []
      for src, r, idx, _ in grp:
        if idx is None:
          vals.append(src[r, cols])
        else:
          vals.append(plsc.load_gather(src, [jnp.full((L,), r, jnp.int32), idx]))
      for (_, _, _, orow), v in zip(grp, vals):
        out_v[orow, cols] = v
    return carry

  lax.fori_loop(0, NCHUNK, chunk, 0)
  pltpu.sync_copy(out_v, out_h.at[:, pl.ds(base, BPW)])


def kernel(continuous, binary, controller, action, jumps_left, character,
           l_cancel, hurtbox_state, ground, last_attack_landed,
           W_action, W_jumps, W_character, W_l_cancel, W_hurtbox, W_ground,
           W_last_attack):
  to_i32 = lambda x: x.astype(jnp.int32)
  out_t = _encode(continuous.T, binary.T, controller.T,
                  to_i32(action), to_i32(jumps_left), to_i32(character),
                  to_i32(l_cancel), to_i32(hurtbox_state), to_i32(ground),
                  to_i32(last_attack_landed),
                  W_action.T, W_jumps.T, W_character.T, W_l_cancel.T,
                  W_hurtbox.T, W_ground.T, W_last_attack.T)
  return out_t.T


# speedup vs baseline: 4.1582x; 1.0162x over previous
import functools

import jax
import jax.numpy as jnp
from jax import lax
from jax.experimental import pallas as pl
from jax.experimental.pallas import tpu as pltpu
from jax.experimental.pallas import tpu_sc as plsc

NC = 2
NS = 16
L = 16
NW = NC * NS

B = 16384
BPW = B // NW
HALF = BPW // 2
NCHUNK_H = HALF // L

CONT_PARTS = ((13, 0), (3, 13), (13, 16))
EMB_PARTS = ((32, 29), (4, 61), (8, 65), (2, 73),
             (2, 75), (4, 77), (8, 81))
D_OUT = 89

_TABLE_SHAPES_T = ((32, 400), (4, 8), (8, 33), (2, 3), (2, 3), (4, 32), (8, 64))

_mesh = plsc.VectorSubcoreMesh(
    core_axis_name="c", subcore_axis_name="s", num_cores=NC, num_subcores=NS)


@functools.partial(
    pl.kernel,
    mesh=_mesh,
    compiler_params=pltpu.CompilerParams(
        needs_layout_passes=False, use_tc_tiling_on_sc=True),
    out_type=jax.ShapeDtypeStruct((D_OUT, B), jnp.float32),
    scratch_types=(
        [pltpu.VMEM((2, 13, HALF), jnp.float32),
         pltpu.VMEM((2, 3, HALF), jnp.float32),
         pltpu.VMEM((2, 13, HALF), jnp.float32)]
        + [pltpu.VMEM((BPW,), jnp.int32) for _ in range(7)]
        + [pltpu.VMEM(s, jnp.float32) for s in _TABLE_SHAPES_T]
        + [pltpu.VMEM((2, D_OUT, HALF), jnp.float32)]
        + [pltpu.SemaphoreType.DMA, pltpu.SemaphoreType.DMA,
           pltpu.SemaphoreType.DMA]
    ),
)
def _encode(cont_h, bin_h, ctrl_h,
            act_h, jmp_h, chr_h, lc_h, hb_h, gnd_h, la_h,
            wa_h, wj_h, wc_h, wl_h, wh_h, wg_h, wla_h,
            out_h,
            cont_v, bin_v, ctrl_v,
            act_v, jmp_v, chr_v, lc_v, hb_v, gnd_v, la_v,
            wa_v, wj_v, wc_v, wl_v, wh_v, wg_v, wla_v,
            out_v, sem_a, sem_b, sem_out):
  wid = lax.axis_index("s") * NC + lax.axis_index("c")
  base = wid * BPW

  idx_refs = (act_v, jmp_v, chr_v, lc_v, hb_v, gnd_v, la_v)
  tbl_refs = (wa_v, wj_v, wc_v, wl_v, wh_v, wg_v, wla_v)
  in_hbm = (cont_h, bin_h, ctrl_h)
  in_vmem = (cont_v, bin_v, ctrl_v)

  def fire_half(hh, sem):
    return [pltpu.async_copy(h.at[:, pl.ds(base + hh * HALF, HALF)],
                             v.at[hh], sem)
            for h, v in zip(in_hbm, in_vmem)]

  batch_a = fire_half(0, sem_a)
  for h, v in zip((act_h, jmp_h, chr_h, lc_h, hb_h, gnd_h, la_h), idx_refs):
    batch_a.append(pltpu.async_copy(h.at[pl.ds(base, BPW)], v, sem_a))
  for h, v in zip((wa_h, wj_h, wc_h, wl_h, wh_h, wg_h, wla_h), tbl_refs):
    batch_a.append(pltpu.async_copy(h, v, sem_a))
  batch_b = fire_half(1, sem_b)

  GRP = 8

  def compute_half(hh):
    def chunk(k, carry):
      cols = pl.ds(k * L, L)
      jobs = []
      for src, (w, off) in zip(in_vmem, CONT_PARTS):
        for r in range(w):
          jobs.append((src, r, None, off + r))
      for iv, tv, (w, off) in zip(idx_refs, tbl_refs, EMB_PARTS):
        idx = iv[pl.ds(hh * HALF + k * L, L)]
        for r in range(w):
          jobs.append((tv, r, idx, off + r))
      for g in range(0, len(jobs), GRP):
        grp = jobs[g:g + GRP]
        vals = []
        for src, r, idx, _ in grp:
          if idx is None:
            vals.append(src[hh, r, cols])
          else:
            vals.append(
                plsc.load_gather(src, [jnp.full((L,), r, jnp.int32), idx]))
        for (_, _, _, orow), v in zip(grp, vals):
          out_v[hh, orow, cols] = v
      return carry

    lax.fori_loop(0, NCHUNK_H, chunk, 0)

  for c in batch_a:
    c.wait()
  compute_half(0)
  out0 = pltpu.async_copy(
      out_v.at[0], out_h.at[:, pl.ds(base, HALF)], sem_out)
  for c in batch_b:
    c.wait()
  compute_half(1)
  out1 = pltpu.async_copy(
      out_v.at[1], out_h.at[:, pl.ds(base + HALF, HALF)], sem_out)
  out0.wait()
  out1.wait()


def kernel(continuous, binary, controller, action, jumps_left, character,
           l_cancel, hurtbox_state, ground, last_attack_landed,
           W_action, W_jumps, W_character, W_l_cancel, W_hurtbox, W_ground,
           W_last_attack):
  to_i32 = lambda x: x.astype(jnp.int32)
  out_t = _encode(continuous.T, binary.T, controller.T,
                  to_i32(action), to_i32(jumps_left), to_i32(character),
                  to_i32(l_cancel), to_i32(hurtbox_state), to_i32(ground),
                  to_i32(last_attack_landed),
                  W_action.T, W_jumps.T, W_character.T, W_l_cancel.T,
                  W_hurtbox.T, W_ground.T, W_last_attack.T)
  return out_t.T
